# baseline (speedup 1.0000x reference)
import functools

import jax
import jax.numpy as jnp
from jax import lax
from jax.experimental import pallas as pl
from jax.experimental.pallas import tpu as pltpu
from jax.experimental.pallas import tpu_sc as plsc

B = 16384
F = 32
NC = 2
NS = 16
NW = NC * NS
BPW = B // NW
CHUNK = 128
NCHUNK = BPW // CHUNK


def _gmf_body(ut_hbm, it_hbm, uidx_hbm, iidx_hbm, aux_hbm, out_hbm,
              uidx_v, iidx_v, urows, irows, aux_v, out_v, sem):
    wid = lax.axis_index("s") * NC + lax.axis_index("c")

    pltpu.sync_copy(uidx_hbm.at[wid], uidx_v)
    pltpu.sync_copy(iidx_hbm.at[wid], iidx_v)
    pltpu.sync_copy(aux_hbm, aux_v)

    copies = []
    for c in range(NCHUNK):
        dst = urows.at[pl.ds(c * CHUNK, CHUNK)]
        copies.append(pltpu.async_copy(ut_hbm.at[uidx_v.at[c]], dst, sem))
        dst = irows.at[pl.ds(c * CHUNK, CHUNK)]
        copies.append(pltpu.async_copy(it_hbm.at[iidx_v.at[c]], dst, sem))
    for cp in copies:
        cp.wait()

    lane = lax.iota(jnp.int32, 16)
    bvec = aux_v[F]

    @pl.loop(0, BPW // 16)
    def _group(g):
        ridx = g * 16 + lane
        acc = bvec
        for f in range(F):
            cf = jnp.full((16,), f, jnp.int32)
            u = plsc.load_gather(urows, [ridx, cf])
            it = plsc.load_gather(irows, [ridx, cf])
            acc = acc + aux_v[f] * u * it
        y = 1.0 / (1.0 + jnp.exp(-acc))
        out_v[pl.ds(g * 16, 16)] = y

    pltpu.sync_copy(out_v, out_hbm.at[pl.ds(wid * BPW, BPW)])


@jax.jit
def _gmf_sc(user_table, item_table, uidx, iidx, aux):
    mesh = plsc.VectorSubcoreMesh(core_axis_name="c", subcore_axis_name="s")
    fn = pl.kernel(
        _gmf_body,
        out_type=jax.ShapeDtypeStruct((B,), jnp.float32),
        mesh=mesh,
        scratch_types=[
            pltpu.VMEM((NCHUNK, CHUNK), jnp.int32),
            pltpu.VMEM((NCHUNK, CHUNK), jnp.int32),
            pltpu.VMEM((BPW, F), jnp.float32),
            pltpu.VMEM((BPW, F), jnp.float32),
            pltpu.VMEM((F + 1, 16), jnp.float32),
            pltpu.VMEM((BPW,), jnp.float32),
            pltpu.SemaphoreType.DMA,
        ],
        compiler_params=pltpu.CompilerParams(
            needs_layout_passes=False, use_tc_tiling_on_sc=False),
    )
    return fn(user_table, item_table, uidx, iidx, aux)


def kernel(users, items, user_table, item_table, predict_w, predict_b):
    uidx = users.astype(jnp.int32).reshape(NW, NCHUNK, CHUNK)
    iidx = items.astype(jnp.int32).reshape(NW, NCHUNK, CHUNK)
    w_bcast = jnp.broadcast_to(predict_w.reshape(F, 1), (F, 16))
    b_bcast = jnp.broadcast_to(predict_b.reshape(1, 1), (1, 16))
    aux = jnp.concatenate([w_bcast, b_bcast], axis=0).astype(jnp.float32)
    out = _gmf_sc(user_table, item_table, uidx, iidx, aux)
    return out.reshape(B, 1)
